# baseline (speedup 1.0000x reference)
import functools

import jax
import jax.numpy as jnp
from jax.experimental import pallas as pl
from jax.experimental.pallas import tpu as pltpu

_TOKENS_PER_TILE = 4096
_LANES = 128
_CORES = 2


def _round_up(x, m):
    return (x + m - 1) // m * m


def _gather_tile_kernel(ids_ref, table_hbm, o_ref, tab_raw, tab_view, sem,
                        *, tokens, p, steps_per_core, vocab):
    core = pl.program_id(0)
    j = pl.program_id(1)

    @pl.when(j == 0)
    def _load_table():
        n_chunks = 4
        rows = vocab // n_chunks
        copies = []
        for k in range(n_chunks):
            cp = pltpu.make_async_copy(
                table_hbm.at[pl.ds(k * rows, rows), :],
                tab_raw.at[pl.ds(k * rows, rows), :],
                sem.at[k],
            )
            cp.start()
            copies.append(cp)

        rows_per = 8 * p
        unroll = 8
        groups_per_chunk = rows // 8 // unroll

        def relayout(g0, _):
            for u in range(unroll):
                g = g0 * unroll + u
                chunk = tab_raw[pl.ds(8 * g, 8), :]
                tab_view[pl.ds(rows_per * g, rows_per), :] = chunk.reshape(
                    rows_per, _LANES)
            return 0

        for k in range(n_chunks):
            copies[k].wait()
            jax.lax.fori_loop(k * groups_per_chunk, (k + 1) * groups_per_chunk,
                              relayout, 0)

    base = (core * steps_per_core + j) * tokens
    for mi in range(tokens):
        idx = pl.multiple_of(ids_ref[base + mi], p)
        slab = tab_view[pl.ds(idx, p), :]
        o_ref[mi, :] = slab.reshape(p * _LANES)


def kernel(sentence, embed_table):
    B, S = sentence.shape
    V, E = embed_table.shape
    T = _TOKENS_PER_TILE
    p = E // _LANES

    flat = sentence.reshape(-1).astype(jnp.int32)
    N = flat.shape[0]
    N_pad = _round_up(N, T * _CORES)
    if N_pad != N:
        flat = jnp.pad(flat, (0, N_pad - N))
    ids = flat * p

    steps_per_core = N_pad // (T * _CORES)
    grid = (_CORES, steps_per_core)

    vmem_bytes = 2 * V * E * 4 + 4 * T * E * 4 + (4 << 20)

    out = pl.pallas_call(
        functools.partial(_gather_tile_kernel, tokens=T, p=p,
                          steps_per_core=steps_per_core, vocab=V),
        out_shape=jax.ShapeDtypeStruct((N_pad, E), embed_table.dtype),
        grid_spec=pltpu.PrefetchScalarGridSpec(
            num_scalar_prefetch=1,
            grid=grid,
            in_specs=[
                pl.BlockSpec(memory_space=pl.ANY),
            ],
            out_specs=pl.BlockSpec(
                (T, E), lambda i, j, ids, spc=steps_per_core: (i * spc + j, 0)
            ),
            scratch_shapes=[
                pltpu.VMEM((V, E), embed_table.dtype),
                pltpu.VMEM((V * p, _LANES), embed_table.dtype),
                pltpu.SemaphoreType.DMA((4,)),
            ],
        ),
        compiler_params=pltpu.CompilerParams(
            dimension_semantics=("parallel", "arbitrary"),
            vmem_limit_bytes=vmem_bytes,
        ),
    )(ids, embed_table)

    return out[:N].reshape(B, S, E)


# speedup vs baseline: 1.0647x; 1.0647x over previous
import functools

import jax
import jax.numpy as jnp
from jax.experimental import pallas as pl
from jax.experimental.pallas import tpu as pltpu

_TOKENS_PER_TILE = 2048
_LANES = 128
_CORES = 2


def _round_up(x, m):
    return (x + m - 1) // m * m


def _gather_tile_kernel(ids_ref, table_hbm, o_ref, tab_raw, tab_view, sem,
                        *, tokens, p, steps_per_core, vocab):
    core = pl.program_id(0)
    j = pl.program_id(1)

    @pl.when(j == 0)
    def _load_table():
        n_chunks = 4
        rows = vocab // n_chunks
        copies = []
        for k in range(n_chunks):
            cp = pltpu.make_async_copy(
                table_hbm.at[pl.ds(k * rows, rows), :],
                tab_raw.at[pl.ds(k * rows, rows), :],
                sem.at[k],
            )
            cp.start()
            copies.append(cp)

        rows_per = 8 * p
        unroll = 8
        groups_per_chunk = rows // 8 // unroll

        def relayout(g0, _):
            for u in range(unroll):
                g = g0 * unroll + u
                chunk = tab_raw[pl.ds(8 * g, 8), :]
                tab_view[pl.ds(rows_per * g, rows_per), :] = chunk.reshape(
                    rows_per, _LANES)
            return 0

        for k in range(n_chunks):
            copies[k].wait()
            jax.lax.fori_loop(k * groups_per_chunk, (k + 1) * groups_per_chunk,
                              relayout, 0)

    base = (core * steps_per_core + j) * tokens
    for mi in range(tokens):
        idx = pl.multiple_of(ids_ref[base + mi], p)
        slab = tab_view[pl.ds(idx, p), :]
        o_ref[mi, :] = slab.reshape(p * _LANES)


def kernel(sentence, embed_table):
    B, S = sentence.shape
    V, E = embed_table.shape
    T = _TOKENS_PER_TILE
    p = E // _LANES

    flat = sentence.reshape(-1).astype(jnp.int32)
    N = flat.shape[0]
    N_pad = _round_up(N, T * _CORES)
    if N_pad != N:
        flat = jnp.pad(flat, (0, N_pad - N))
    ids = flat * p

    steps_per_core = N_pad // (T * _CORES)
    grid = (_CORES, steps_per_core)

    vmem_bytes = 2 * V * E * 4 + 4 * T * E * 4 + (4 << 20)

    out = pl.pallas_call(
        functools.partial(_gather_tile_kernel, tokens=T, p=p,
                          steps_per_core=steps_per_core, vocab=V),
        out_shape=jax.ShapeDtypeStruct((N_pad, E), embed_table.dtype),
        grid_spec=pltpu.PrefetchScalarGridSpec(
            num_scalar_prefetch=1,
            grid=grid,
            in_specs=[
                pl.BlockSpec(memory_space=pl.ANY),
            ],
            out_specs=pl.BlockSpec(
                (T, E), lambda i, j, ids, spc=steps_per_core: (i * spc + j, 0)
            ),
            scratch_shapes=[
                pltpu.VMEM((V, E), embed_table.dtype),
                pltpu.VMEM((V * p, _LANES), embed_table.dtype),
                pltpu.SemaphoreType.DMA((4,)),
            ],
        ),
        compiler_params=pltpu.CompilerParams(
            dimension_semantics=("parallel", "arbitrary"),
            vmem_limit_bytes=vmem_bytes,
        ),
    )(ids, embed_table)

    return out[:N].reshape(B, S, E)
